# Initial kernel scaffold; baseline (speedup 1.0000x reference)
import functools

import jax
import jax.numpy as jnp
from jax import lax
from jax.experimental import pallas as pl
from jax.experimental.pallas import tpu as pltpu
from jax.experimental.pallas import tpu_sc as plsc

N = 10000
NP = 10016
E = 320000
NG = 64
F_IN = 128
H = 16

NC = 2
NS = 16
NW = NC * NS
CB = 128
NCH = 79
EP = NW * NCH * CB
RPT = NP // NS
TRASH = 10008

_mesh = plsc.VectorSubcoreMesh(core_axis_name="c", subcore_axis_name="s")
_f32 = jnp.float32


@functools.partial(
    pl.kernel,
    out_type=jax.ShapeDtypeStruct((NC, NP, H), _f32),
    mesh=_mesh,
    scratch_types=[
        pltpu.VMEM((NCH, CB), jnp.int32),
        pltpu.VMEM((CB, H), _f32),
        pltpu.VMEM((RPT, H), _f32),
        pltpu.VMEM_SHARED((NP, H), _f32),
        pltpu.SemaphoreType.DMA,
    ],
)
def _sc_degree(dst3, zeros_hbm, ones_hbm, out, didx, onesb, stage, acc_sh, sem):
    cid = lax.axis_index("c")
    sid = lax.axis_index("s")
    wid = sid * NC + cid
    row0 = sid * RPT
    pltpu.sync_copy(zeros_hbm.at[pl.ds(row0, RPT)], stage)
    pltpu.sync_copy(stage, acc_sh.at[pl.ds(row0, RPT)])
    pltpu.sync_copy(dst3.at[wid], didx)
    pltpu.sync_copy(ones_hbm, onesb)
    plsc.subcore_barrier()

    def body(j, carry):
        pltpu.sync_copy(onesb, acc_sh.at[didx.at[j]], add=True)
        return carry

    lax.fori_loop(0, NCH, body, 0)
    plsc.subcore_barrier()
    pltpu.sync_copy(acc_sh.at[pl.ds(row0, RPT)], stage)
    pltpu.sync_copy(stage, out.at[cid, pl.ds(row0, RPT)])


@functools.partial(
    pl.kernel,
    out_type=jax.ShapeDtypeStruct((NC, NP, H), _f32),
    mesh=_mesh,
    scratch_types=[
        pltpu.VMEM((NCH, CB), jnp.int32),
        pltpu.VMEM((NCH, CB), jnp.int32),
        pltpu.VMEM((CB, H), _f32),
        pltpu.VMEM((RPT, H), _f32),
        pltpu.VMEM_SHARED((NP, H), _f32),
        pltpu.VMEM_SHARED((NP, H), _f32),
        pltpu.SemaphoreType.DMA,
    ],
)
def _sc_scatter(src3, dst3, g_hbm, zeros_hbm, out,
                sidx, didx, rbuf, stage, g_sh, acc_sh, sem):
    cid = lax.axis_index("c")
    sid = lax.axis_index("s")
    wid = sid * NC + cid
    row0 = sid * RPT
    pltpu.sync_copy(zeros_hbm.at[pl.ds(row0, RPT)], stage)
    pltpu.sync_copy(stage, acc_sh.at[pl.ds(row0, RPT)])
    pltpu.sync_copy(g_hbm.at[pl.ds(row0, RPT)], stage)
    pltpu.sync_copy(stage, g_sh.at[pl.ds(row0, RPT)])
    pltpu.sync_copy(src3.at[wid], sidx)
    pltpu.sync_copy(dst3.at[wid], didx)
    plsc.subcore_barrier()

    def body(j, carry):
        pltpu.async_copy(g_sh.at[sidx.at[j]], rbuf, sem).wait()
        pltpu.sync_copy(rbuf, acc_sh.at[didx.at[j]], add=True)
        return carry

    lax.fori_loop(0, NCH, body, 0)
    plsc.subcore_barrier()
    pltpu.sync_copy(acc_sh.at[pl.ds(row0, RPT)], stage)
    pltpu.sync_copy(stage, out.at[cid, pl.ds(row0, RPT)])


def _tc1_body(x_ref, w1_ref, da_ref, db_ref, h1_ref, g1_ref, di_ref):
    deg = da_ref[:, 0:1] + db_ref[:, 0:1] + 1.0
    di16 = jnp.broadcast_to(lax.rsqrt(deg), (NP, H))
    h1 = jnp.dot(x_ref[...], w1_ref[...], preferred_element_type=_f32)
    h1_ref[...] = h1
    g1_ref[...] = di16 * h1
    di_ref[...] = di16


def _tc2_body(aa_ref, ab_ref, h1_ref, di_ref, b1_ref, w2_ref, h2_ref, g2_ref):
    di16 = di_ref[...]
    r = jnp.maximum(
        di16 * (aa_ref[...] + ab_ref[...]) + di16 * di16 * h1_ref[...]
        + b1_ref[...], 0.0)
    h2 = jnp.dot(r, w2_ref[...], preferred_element_type=_f32)
    h2_ref[...] = h2
    g2_ref[...] = di16 * h2


def _tc3_body(aa_ref, ab_ref, h2_ref, di_ref, b2_ref, batch_ref, wfc_ref,
              bfc_ref, y_ref):
    di16 = di_ref[...]
    out2 = (di16 * (aa_ref[...] + ab_ref[...])
            + di16 * di16 * h2_ref[...] + b2_ref[...])
    gids = lax.broadcasted_iota(jnp.int32, (1, NG), 1)
    oh = (batch_ref[...] == gids).astype(_f32)
    sums = lax.dot_general(oh, out2, (((0,), (0,)), ((), ())),
                           preferred_element_type=_f32)
    cnt = jnp.sum(oh, axis=0)[:, None]
    pooled = sums / jnp.maximum(cnt, 1.0)
    y_ref[...] = jnp.dot(pooled, wfc_ref[...], preferred_element_type=_f32) \
        + bfc_ref[...]


_tc1 = pl.pallas_call(
    _tc1_body,
    out_shape=[jax.ShapeDtypeStruct((NP, H), _f32)] * 3,
)

_tc2 = pl.pallas_call(
    _tc2_body,
    out_shape=[jax.ShapeDtypeStruct((NP, H), _f32)] * 2,
)

_tc3 = pl.pallas_call(
    _tc3_body,
    out_shape=jax.ShapeDtypeStruct((NG, 1), _f32),
)


def kernel(x, edge_index, batch, W1, b1, W2, b2, Wfc, bfc):
    ei = edge_index.astype(jnp.int32)
    src = jnp.concatenate(
        [ei[0], jnp.zeros((EP - E,), jnp.int32)]).reshape(NW, NCH, CB)
    dst = jnp.concatenate(
        [ei[1], jnp.full((EP - E,), TRASH, jnp.int32)]).reshape(NW, NCH, CB)
    xp = jnp.pad(x, ((0, NP - N), (0, 0)))
    batch2 = jnp.concatenate(
        [batch.astype(jnp.int32), jnp.full((NP - N,), -1, jnp.int32)]
    ).reshape(NP, 1)
    zeros = jnp.zeros((NP, H), _f32)
    ones_rows = jnp.ones((CB, H), _f32)

    degp = _sc_degree(dst, zeros, ones_rows)
    h1, g1, di16 = _tc1(xp, W1, degp[0], degp[1])
    acc1 = _sc_scatter(src, dst, g1, zeros)
    h2, g2 = _tc2(acc1[0], acc1[1], h1, di16, b1.reshape(1, H), W2)
    acc2 = _sc_scatter(src, dst, g2, zeros)
    y = _tc3(acc2[0], acc2[1], h2, di16, b2.reshape(1, H), batch2,
             Wfc, bfc.reshape(1, 1))
    return y


# speedup vs baseline: 50.7088x; 50.7088x over previous
import functools

import jax
import jax.numpy as jnp
from jax import lax
from jax.experimental import pallas as pl
from jax.experimental.pallas import tpu as pltpu
from jax.experimental.pallas import tpu_sc as plsc

N = 10000
NP = 10112
E = 320000
NG = 64
F_IN = 128
H = 16

NC = 2
NS = 16
NW = NC * NS
CB = 128
NCH = 80
EP = NW * NCH * CB
RPT = NP // NS

_mesh = plsc.VectorSubcoreMesh(core_axis_name="c", subcore_axis_name="s")
_params = pltpu.CompilerParams(use_tc_tiling_on_sc=False)
_f32 = jnp.float32


@functools.partial(
    pl.kernel,
    out_type=jax.ShapeDtypeStruct((NC, NP, H), _f32),
    mesh=_mesh,
    compiler_params=_params,
    scratch_types=[
        pltpu.VMEM((NCH, CB), jnp.int32),
        pltpu.VMEM((CB, H), _f32),
        pltpu.VMEM_SHARED((NP, H), _f32),
        pltpu.SemaphoreType.DMA,
    ],
)
def _sc_degree(dst3, zeros_hbm, ones_hbm, out, didx, onesb, acc_sh, sem):
    cid = lax.axis_index("c")
    sid = lax.axis_index("s")
    wid = sid * NC + cid
    row0 = sid * RPT
    pltpu.sync_copy(zeros_hbm.at[pl.ds(row0, RPT)], acc_sh.at[pl.ds(row0, RPT)])
    pltpu.sync_copy(dst3.at[wid], didx)
    pltpu.sync_copy(ones_hbm, onesb)
    plsc.subcore_barrier()

    def body(j, carry):
        pltpu.sync_copy(onesb, acc_sh.at[didx.at[j]], add=True)
        return carry

    lax.fori_loop(0, NCH, body, 0)
    plsc.subcore_barrier()
    pltpu.sync_copy(acc_sh.at[pl.ds(row0, RPT)], out.at[cid, pl.ds(row0, RPT)])


@functools.partial(
    pl.kernel,
    out_type=jax.ShapeDtypeStruct((NC, NP, H), _f32),
    mesh=_mesh,
    compiler_params=_params,
    scratch_types=[
        pltpu.VMEM((NCH, CB), jnp.int32),
        pltpu.VMEM((NCH, CB), jnp.int32),
        pltpu.VMEM((CB, H), _f32),
        pltpu.VMEM((CB, H), _f32),
        pltpu.VMEM_SHARED((NP, H), _f32),
        pltpu.VMEM_SHARED((NP, H), _f32),
        pltpu.SemaphoreType.DMA,
        pltpu.SemaphoreType.DMA,
    ],
)
def _sc_scatter(src3, dst3, g_hbm, zeros_hbm, out,
                sidx, didx, bufa, bufb, g_sh, acc_sh, sema, semb):
    cid = lax.axis_index("c")
    sid = lax.axis_index("s")
    wid = sid * NC + cid
    row0 = sid * RPT
    pltpu.sync_copy(zeros_hbm.at[pl.ds(row0, RPT)], acc_sh.at[pl.ds(row0, RPT)])
    pltpu.sync_copy(g_hbm.at[pl.ds(row0, RPT)], g_sh.at[pl.ds(row0, RPT)])
    pltpu.sync_copy(src3.at[wid], sidx)
    pltpu.sync_copy(dst3.at[wid], didx)
    plsc.subcore_barrier()

    def gather(j, buf, sem):
        pltpu.async_copy(g_sh.at[sidx.at[j]], buf, sem)

    def wait(j, buf, sem):
        pltpu.make_async_copy(g_sh.at[sidx.at[j]], buf, sem).wait()

    def scatter(j, buf):
        pltpu.sync_copy(buf, acc_sh.at[didx.at[j]], add=True)

    gather(0, bufa, sema)

    def body(k, carry):
        ja = 2 * k
        gather(ja + 1, bufb, semb)
        wait(ja, bufa, sema)
        scatter(ja, bufa)

        @pl.when(ja + 2 < NCH)
        def _():
            gather(ja + 2, bufa, sema)

        wait(ja + 1, bufb, semb)
        scatter(ja + 1, bufb)
        return carry

    lax.fori_loop(0, NCH // 2, body, 0)
    plsc.subcore_barrier()
    pltpu.sync_copy(acc_sh.at[pl.ds(row0, RPT)], out.at[cid, pl.ds(row0, RPT)])


def _tc1_body(x_ref, w1_ref, da_ref, db_ref, h1_ref, g1_ref, di_ref):
    deg = da_ref[:, 0:1] + db_ref[:, 0:1] + 1.0
    di16 = jnp.broadcast_to(lax.rsqrt(deg), (NP, H))
    h1 = jnp.dot(x_ref[...], w1_ref[...], preferred_element_type=_f32)
    h1_ref[...] = h1
    g1_ref[...] = di16 * h1
    di_ref[...] = di16


def _tc2_body(aa_ref, ab_ref, h1_ref, di_ref, b1_ref, w2_ref, h2_ref, g2_ref):
    di16 = di_ref[...]
    r = jnp.maximum(
        di16 * (aa_ref[...] + ab_ref[...]) + di16 * di16 * h1_ref[...]
        + b1_ref[...], 0.0)
    h2 = jnp.dot(r, w2_ref[...], preferred_element_type=_f32)
    h2_ref[...] = h2
    g2_ref[...] = di16 * h2


def _tc3_body(aa_ref, ab_ref, h2_ref, di_ref, b2_ref, batch_ref, wfc_ref,
              bfc_ref, y_ref):
    di16 = di_ref[...]
    out2 = (di16 * (aa_ref[...] + ab_ref[...])
            + di16 * di16 * h2_ref[...] + b2_ref[...])
    gids = lax.broadcasted_iota(jnp.int32, (1, NG), 1)
    oh = (batch_ref[...] == gids).astype(_f32)
    sums = lax.dot_general(oh, out2, (((0,), (0,)), ((), ())),
                           preferred_element_type=_f32)
    cnt = jnp.sum(oh, axis=0)[:, None]
    pooled = sums / jnp.maximum(cnt, 1.0)
    y_ref[...] = jnp.dot(pooled, wfc_ref[...], preferred_element_type=_f32) \
        + bfc_ref[...]


_tc1 = pl.pallas_call(
    _tc1_body,
    out_shape=[jax.ShapeDtypeStruct((NP, H), _f32)] * 3,
)

_tc2 = pl.pallas_call(
    _tc2_body,
    out_shape=[jax.ShapeDtypeStruct((NP, H), _f32)] * 2,
)

_tc3 = pl.pallas_call(
    _tc3_body,
    out_shape=jax.ShapeDtypeStruct((NG, 1), _f32),
)


def kernel(x, edge_index, batch, W1, b1, W2, b2, Wfc, bfc):
    ei = edge_index.astype(jnp.int32)
    trash = N + jnp.arange(EP - E, dtype=jnp.int32) % (NP - N)
    src = jnp.concatenate(
        [ei[0], jnp.zeros((EP - E,), jnp.int32)]).reshape(NW, NCH, CB)
    dst = jnp.concatenate([ei[1], trash]).reshape(NW, NCH, CB)
    xp = jnp.pad(x, ((0, NP - N), (0, 0)))
    batch2 = jnp.concatenate(
        [batch.astype(jnp.int32), jnp.full((NP - N,), -1, jnp.int32)]
    ).reshape(NP, 1)
    zeros = jnp.zeros((NP, H), _f32)
    ones_rows = jnp.ones((CB, H), _f32)

    degp = _sc_degree(dst, zeros, ones_rows)
    h1, g1, di16 = _tc1(xp, W1, degp[0], degp[1])
    acc1 = _sc_scatter(src, dst, g1, zeros)
    h2, g2 = _tc2(acc1[0], acc1[1], h1, di16, b1.reshape(1, H), W2)
    acc2 = _sc_scatter(src, dst, g2, zeros)
    y = _tc3(acc2[0], acc2[1], h2, di16, b2.reshape(1, H), batch2,
             Wfc, bfc.reshape(1, 1))
    return y


# speedup vs baseline: 83.3152x; 1.6430x over previous
import functools

import jax
import jax.numpy as jnp
from jax import lax
from jax.experimental import pallas as pl
from jax.experimental.pallas import tpu as pltpu
from jax.experimental.pallas import tpu_sc as plsc

N = 10000
NP = 10112
E = 320000
NG = 64
F_IN = 128
H = 16

NC = 2
NS = 16
NW = NC * NS
CB = 128
NCH = 80
EP = NW * NCH * CB
RPT = NP // NS

_mesh = plsc.VectorSubcoreMesh(core_axis_name="c", subcore_axis_name="s")
_params = pltpu.CompilerParams(use_tc_tiling_on_sc=False)
_f32 = jnp.float32


@functools.partial(
    pl.kernel,
    out_type=jax.ShapeDtypeStruct((NC, NP, H), _f32),
    mesh=_mesh,
    compiler_params=_params,
    scratch_types=[
        pltpu.VMEM((NCH, CB), jnp.int32),
        pltpu.VMEM((CB, H), _f32),
        pltpu.VMEM_SHARED((NP, H), _f32),
        pltpu.SemaphoreType.DMA,
    ],
)
def _sc_degree(dst3, zeros_hbm, ones_hbm, out, didx, onesb, acc_sh, sem):
    cid = lax.axis_index("c")
    sid = lax.axis_index("s")
    wid = sid * NC + cid
    row0 = sid * RPT
    pltpu.sync_copy(zeros_hbm.at[pl.ds(row0, RPT)], acc_sh.at[pl.ds(row0, RPT)])
    pltpu.sync_copy(dst3.at[wid], didx)
    pltpu.sync_copy(ones_hbm, onesb)
    plsc.subcore_barrier()

    def body(j, carry):
        pltpu.sync_copy(onesb, acc_sh.at[didx.at[j]], add=True)
        return carry

    lax.fori_loop(0, NCH, body, 0)
    plsc.subcore_barrier()
    pltpu.sync_copy(acc_sh.at[pl.ds(row0, RPT)], out.at[cid, pl.ds(row0, RPT)])


@functools.partial(
    pl.kernel,
    out_type=jax.ShapeDtypeStruct((NC, NP, H), _f32),
    mesh=_mesh,
    compiler_params=_params,
    scratch_types=[
        pltpu.VMEM((NCH, CB), jnp.int32),
        pltpu.VMEM((NCH, CB), jnp.int32),
        pltpu.VMEM((CB, H), _f32),
        pltpu.VMEM((CB, H), _f32),
        pltpu.VMEM_SHARED((NP, H), _f32),
        pltpu.VMEM_SHARED((NP, H), _f32),
        pltpu.SemaphoreType.DMA,
        pltpu.SemaphoreType.DMA,
    ],
)
def _sc_scatter(src3, dst3, g_hbm, zeros_hbm, out,
                sidx, didx, bufa, bufb, g_sh, acc_sh, sema, semb):
    cid = lax.axis_index("c")
    sid = lax.axis_index("s")
    wid = sid * NC + cid
    row0 = sid * RPT
    pltpu.sync_copy(zeros_hbm.at[pl.ds(row0, RPT)], acc_sh.at[pl.ds(row0, RPT)])
    pltpu.sync_copy(g_hbm.at[pl.ds(row0, RPT)], g_sh.at[pl.ds(row0, RPT)])
    pltpu.sync_copy(src3.at[wid], sidx)
    pltpu.sync_copy(dst3.at[wid], didx)
    plsc.subcore_barrier()

    def gather(j, buf, sem):
        pltpu.async_copy(g_sh.at[sidx.at[j]], buf, sem)

    def wait(j, buf, sem):
        pltpu.make_async_copy(g_sh.at[sidx.at[j]], buf, sem).wait()

    def scatter(j, buf):
        pltpu.sync_copy(buf, acc_sh.at[didx.at[j]], add=True)

    gather(0, bufa, sema)

    def body(k, carry):
        ja = 2 * k
        gather(ja + 1, bufb, semb)
        wait(ja, bufa, sema)
        scatter(ja, bufa)

        @pl.when(ja + 2 < NCH)
        def _():
            gather(ja + 2, bufa, sema)

        wait(ja + 1, bufb, semb)
        scatter(ja + 1, bufb)
        return carry

    lax.fori_loop(0, NCH // 2, body, 0)
    plsc.subcore_barrier()
    pltpu.sync_copy(acc_sh.at[pl.ds(row0, RPT)], out.at[cid, pl.ds(row0, RPT)])


PR = NP * H // 128
PRN = N * H // 128


def _tc1_body(x8_ref, w1bd_ref, dp_ref, h1_ref, g1_ref, di_ref):
    deg = dp_ref[0] + dp_ref[1] + 1.0
    dip = lax.rsqrt(deg)
    h1 = jnp.dot(x8_ref[...], w1bd_ref[...], preferred_element_type=_f32)
    h1_ref[0:PRN] = h1
    h1_ref[PRN:PR] = jnp.zeros((PR - PRN, 128), _f32)
    g1_ref[0:PRN] = dip[0:PRN] * h1
    g1_ref[PRN:PR] = jnp.zeros((PR - PRN, 128), _f32)
    di_ref[...] = dip


def _tc2_body(a1_ref, h1_ref, di_ref, b1t_ref, w2bd_ref, h2_ref, g2_ref):
    dip = di_ref[...]
    r = jnp.maximum(
        dip * (a1_ref[0] + a1_ref[1]) + dip * dip * h1_ref[...]
        + b1t_ref[...], 0.0)
    h2 = jnp.dot(r, w2bd_ref[...], preferred_element_type=_f32)
    h2_ref[...] = h2
    g2_ref[...] = dip * h2


def _tc3_body(a2_ref, h2_ref, di_ref, b2t_ref, batch8_ref, wfc_ref,
              bfc_ref, y_ref):
    dip = di_ref[...]
    out2p = (dip * (a2_ref[0] + a2_ref[1]) + dip * dip * h2_ref[...]
             + b2t_ref[...])[0:PRN]
    gids = lax.broadcasted_iota(jnp.int32, (1, NG), 1)
    ones_col = jnp.ones((PRN, 1), _f32)
    sums = jnp.zeros((NG, H), _f32)
    cnt = jnp.zeros((NG, 1), _f32)
    for k in range(8):
        ohk = (batch8_ref[k][:, None] == gids).astype(_f32)
        part = lax.dot_general(ohk, out2p, (((0,), (0,)), ((), ())),
                               preferred_element_type=_f32)
        sums = sums + part[:, H * k:H * (k + 1)]
        cnt = cnt + lax.dot_general(ohk, ones_col, (((0,), (0,)), ((), ())),
                                    preferred_element_type=_f32)
    pooled = sums / jnp.maximum(cnt, 1.0)
    y_ref[...] = jnp.dot(pooled, wfc_ref[...], preferred_element_type=_f32) \
        + bfc_ref[...]


_tc1 = pl.pallas_call(
    _tc1_body,
    out_shape=[jax.ShapeDtypeStruct((PR, 128), _f32)] * 3,
)

_tc2 = pl.pallas_call(
    _tc2_body,
    out_shape=[jax.ShapeDtypeStruct((PR, 128), _f32)] * 2,
)

_tc3 = pl.pallas_call(
    _tc3_body,
    out_shape=jax.ShapeDtypeStruct((NG, 1), _f32),
)


def _blockdiag8(w):
    k = w.shape[0]
    return (jnp.eye(8, dtype=_f32)[:, None, :, None]
            * w[None, :, None, :]).reshape(8 * k, 8 * H)


def kernel(x, edge_index, batch, W1, b1, W2, b2, Wfc, bfc):
    ei = edge_index.astype(jnp.int32)
    trash = N + jnp.arange(EP - E, dtype=jnp.int32) % (NP - N)
    src = jnp.concatenate(
        [ei[0], jnp.zeros((EP - E,), jnp.int32)]).reshape(NW, NCH, CB)
    dst = jnp.concatenate([ei[1], trash]).reshape(NW, NCH, CB)
    x8 = x.reshape(PRN, 8 * F_IN)
    zeros = jnp.zeros((NP, H), _f32)
    ones_rows = jnp.ones((CB, H), _f32)

    degp = _sc_degree(dst, zeros, ones_rows)
    h1, g1, di = _tc1(x8, _blockdiag8(W1), degp.reshape(NC, PR, 128))
    acc1 = _sc_scatter(src, dst, g1.reshape(NP, H), zeros)
    h2, g2 = _tc2(acc1.reshape(NC, PR, 128), h1, di,
                  jnp.tile(b1, 8).reshape(1, 128), _blockdiag8(W2))
    acc2 = _sc_scatter(src, dst, g2.reshape(NP, H), zeros)
    batch8 = batch.astype(jnp.int32).reshape(PRN, 8).T
    y = _tc3(acc2.reshape(NC, PR, 128), h2, di,
             jnp.tile(b2, 8).reshape(1, 128), batch8,
             Wfc, bfc.reshape(1, 1))
    return y
